# Initial kernel scaffold; baseline (speedup 1.0000x reference)
import jax
import jax.numpy as jnp
from jax.experimental import pallas as pl


def kernel(click_seq, pos_item, neg_item, table, W1, b1, W2, b2, W3, b3):
    raise NotImplementedError("write your pallas kernel here")


# speedup vs baseline: 4.3994x; 4.3994x over previous
import functools

import jax
import jax.numpy as jnp
from jax import lax
from jax.experimental import pallas as pl
from jax.experimental.pallas import tpu as pltpu
from jax.experimental.pallas import tpu_sc as plsc

B, L, V, D = 4096, 50, 100000, 128
H1, H2, H3 = 512, 256, 128
NEG = 4

NC, NS = 2, 16
NW = NC * NS
BPW = B // NW
GPB = 2
GIDX = GPB * L
NJ = BPW // GPB
JR = B // GPB
NCH = D // 16


def _sc_gather_pool(seq_idx, pos_idx, neg_idx, table):
    mesh = plsc.VectorSubcoreMesh(core_axis_name="c", subcore_axis_name="s",
                                  num_cores=NC, num_subcores=NS)

    @functools.partial(
        pl.kernel,
        out_type=(
            jax.ShapeDtypeStruct((B, D), jnp.float32),
            jax.ShapeDtypeStruct((B, D), jnp.float32),
            jax.ShapeDtypeStruct((B * NEG, D), jnp.float32),
        ),
        mesh=mesh,
        scratch_types=[
            pltpu.VMEM((NJ, GIDX), jnp.int32),
            pltpu.VMEM((2, GIDX, D), jnp.float32),
            pltpu.VMEM((BPW, D), jnp.float32),
            pltpu.VMEM((BPW,), jnp.int32),
            pltpu.VMEM((NEG, BPW), jnp.int32),
            pltpu.VMEM((BPW, D), jnp.float32),
            pltpu.SemaphoreType.DMA,
            pltpu.SemaphoreType.DMA,
            pltpu.SemaphoreType.DMA,
        ],
    )
    def k(seq_hbm, pos_hbm, neg_hbm, table_hbm,
          user_hbm, posrows_hbm, negrows_hbm,
          idx_v, rows2_v, user_v, pit_v, nit_v, irows_v,
          sem_a, sem_b, sem_i):
        w = lax.axis_index("s") * NC + lax.axis_index("c")
        jbase = w * NJ
        base = w * BPW

        pltpu.sync_copy(seq_hbm.at[pl.ds(jbase, NJ)], idx_v)
        sems = (sem_a, sem_b)

        def issue(j, p):
            pltpu.async_copy(table_hbm.at[idx_v.at[j]], rows2_v.at[p],
                             sems[p])

        def wait(p):
            pltpu.make_async_copy(table_hbm.at[idx_v.at[0]], rows2_v.at[p],
                                  sems[p]).wait()

        def pool_chunk(j, p):
            rows = rows2_v.at[p]

            def accum(r0, sl):
                def lbody(l, acc):
                    return acc + rows[r0 + l, sl]
                return lax.fori_loop(0, L, lbody,
                                     jnp.zeros((16,), jnp.float32),
                                     unroll=10)

            for b2 in range(GPB):
                for c_i in range(NCH):
                    sl = pl.ds(c_i * 16, 16)
                    user_v[j * GPB + b2, sl] = accum(b2 * L, sl) * (1.0 / L)

        issue(0, 0)
        issue(1, 1)

        def jloop(it, _):
            jj = it * 2
            for p in range(2):
                j = jj + p
                wait(p)
                pool_chunk(j, p)

                @pl.when(j + 2 < NJ)
                def _():
                    issue(j + 2, p)
            return 0

        lax.fori_loop(0, NJ // 2, jloop, 0)
        pltpu.sync_copy(user_v, user_hbm.at[pl.ds(base, BPW)])

        pltpu.sync_copy(pos_hbm.at[w], pit_v)
        pltpu.async_copy(table_hbm.at[pit_v], irows_v, sem_i).wait()
        pltpu.sync_copy(irows_v, posrows_hbm.at[pl.ds(base, BPW)])

        pltpu.sync_copy(neg_hbm.at[pl.ds(w * NEG, NEG)], nit_v)
        for q in range(NEG):
            pltpu.async_copy(table_hbm.at[nit_v.at[q]], irows_v, sem_i).wait()
            pltpu.sync_copy(
                irows_v,
                negrows_hbm.at[pl.ds(w * NEG * BPW + q * BPW, BPW)])

    return k(seq_idx, pos_idx, neg_idx, table)


BM = 512
NB = B // BM


def _tc_body(u0_ref, w1_ref, b1_ref, w2_ref, b2_ref, w3_ref, b3_ref,
             pos_ref, neg_ref, out_ref):
    x = u0_ref[...]
    h = jnp.maximum(
        jnp.dot(x, w1_ref[...], preferred_element_type=jnp.float32)
        + b1_ref[...], 0.0)
    h = jnp.maximum(
        jnp.dot(h, w2_ref[...], preferred_element_type=jnp.float32)
        + b2_ref[...], 0.0)
    u = jnp.maximum(
        jnp.dot(h, w3_ref[...], preferred_element_type=jnp.float32)
        + b3_ref[...], 0.0)

    col = lax.broadcasted_iota(jnp.int32, (D, 1 + NEG), 1)
    s = jnp.dot(u * pos_ref[...], (col == 0).astype(jnp.float32),
                preferred_element_type=jnp.float32)
    for j in range(NEG):
        item = neg_ref[:, j * D:(j + 1) * D]
        s = s + jnp.dot(u * item, (col == (j + 1)).astype(jnp.float32),
                        preferred_element_type=jnp.float32)
    out_ref[...] = s


def _tc_mlp_scores(user0, W1, b1, W2, b2, W3, b3, pos_rows, neg_rows):
    return pl.pallas_call(
        _tc_body,
        grid=(NB,),
        in_specs=[
            pl.BlockSpec((BM, D), lambda i: (i, 0)),
            pl.BlockSpec((D, H1), lambda i: (0, 0)),
            pl.BlockSpec((1, H1), lambda i: (0, 0)),
            pl.BlockSpec((H1, H2), lambda i: (0, 0)),
            pl.BlockSpec((1, H2), lambda i: (0, 0)),
            pl.BlockSpec((H2, H3), lambda i: (0, 0)),
            pl.BlockSpec((1, H3), lambda i: (0, 0)),
            pl.BlockSpec((BM, D), lambda i: (i, 0)),
            pl.BlockSpec((BM, NEG * D), lambda i: (i, 0)),
        ],
        out_specs=pl.BlockSpec((BM, 1 + NEG), lambda i: (i, 0)),
        out_shape=jax.ShapeDtypeStruct((B, 1 + NEG), jnp.float32),
    )(user0, W1, b1.reshape(1, H1), W2, b2.reshape(1, H2),
      W3, b3.reshape(1, H3), pos_rows, neg_rows)


def kernel(click_seq, pos_item, neg_item, table, W1, b1, W2, b2, W3, b3):
    seq_idx = click_seq.astype(jnp.int32).reshape(JR, GIDX)
    pos_idx = pos_item.astype(jnp.int32).reshape(NW, BPW)
    neg_idx = neg_item.astype(jnp.int32).reshape(NW * NEG, BPW)
    user0, pos_rows, neg_rows = _sc_gather_pool(seq_idx, pos_idx, neg_idx,
                                                table)
    return _tc_mlp_scores(user0, W1, b1, W2, b2, W3, b3,
                          pos_rows, neg_rows.reshape(B, NEG * D))


# speedup vs baseline: 5.4524x; 1.2394x over previous
import functools

import jax
import jax.numpy as jnp
from jax import lax
from jax.experimental import pallas as pl
from jax.experimental.pallas import tpu as pltpu
from jax.experimental.pallas import tpu_sc as plsc

B, L, V, D = 4096, 50, 100000, 128
H1, H2, H3 = 512, 256, 128
NEG = 4

NC, NS = 2, 16
NW = NC * NS
BPW = B // NW
GPB = 2
GIDX = GPB * L
NJ = BPW // GPB
JR = B // GPB
NCH = D // 16


def _sc_gather_pool(seq_idx, pos_idx, neg_idx, table):
    mesh = plsc.VectorSubcoreMesh(core_axis_name="c", subcore_axis_name="s",
                                  num_cores=NC, num_subcores=NS)

    @functools.partial(
        pl.kernel,
        out_type=(
            jax.ShapeDtypeStruct((B, D), jnp.float32),
            jax.ShapeDtypeStruct((B, D), jnp.float32),
            jax.ShapeDtypeStruct((B * NEG, D), jnp.float32),
        ),
        mesh=mesh,
        scratch_types=[
            pltpu.VMEM((NJ, GIDX), jnp.int32),
            pltpu.VMEM((2, GIDX, D), jnp.float32),
            pltpu.VMEM((BPW, D), jnp.float32),
            pltpu.VMEM((BPW,), jnp.int32),
            pltpu.VMEM((NEG, BPW), jnp.int32),
            pltpu.VMEM((BPW, D), jnp.float32),
            pltpu.VMEM((2, BPW, D), jnp.float32),
            pltpu.SemaphoreType.DMA,
            pltpu.SemaphoreType.DMA,
            pltpu.SemaphoreType.DMA,
            pltpu.SemaphoreType.DMA,
            pltpu.SemaphoreType.DMA,
        ],
    )
    def k(seq_hbm, pos_hbm, neg_hbm, table_hbm,
          user_hbm, posrows_hbm, negrows_hbm,
          idx_v, rows2_v, user_v, pit_v, nit_v, prows_v, nrows2_v,
          sem_a, sem_b, sem_p, sem_n0, sem_n1):
        w = lax.axis_index("s") * NC + lax.axis_index("c")
        jbase = w * NJ
        base = w * BPW

        pltpu.sync_copy(seq_hbm.at[pl.ds(jbase, NJ)], idx_v)
        pltpu.sync_copy(pos_hbm.at[w], pit_v)
        pltpu.sync_copy(neg_hbm.at[pl.ds(w * NEG, NEG)], nit_v)
        sems = (sem_a, sem_b)
        nsems = (sem_n0, sem_n1)

        def issue(j, p):
            pltpu.async_copy(table_hbm.at[idx_v.at[j]], rows2_v.at[p],
                             sems[p])

        def wait(p):
            pltpu.make_async_copy(table_hbm.at[idx_v.at[0]], rows2_v.at[p],
                                  sems[p]).wait()

        def pool_chunk(j, p):
            rows = rows2_v.at[p]
            for b2 in range(GPB):
                r0 = b2 * L
                accs = [rows[r0, pl.ds(c_i * 16, 16)] for c_i in range(NCH)]
                for l in range(1, L):
                    for c_i in range(NCH):
                        accs[c_i] = accs[c_i] + rows[r0 + l,
                                                     pl.ds(c_i * 16, 16)]
                for c_i in range(NCH):
                    user_v[j * GPB + b2, pl.ds(c_i * 16, 16)] = (
                        accs[c_i] * (1.0 / L))

        issue(0, 0)
        issue(1, 1)
        pltpu.async_copy(table_hbm.at[pit_v], prows_v, sem_p)
        pltpu.async_copy(table_hbm.at[nit_v.at[0]], nrows2_v.at[0], sem_n0)
        pltpu.async_copy(table_hbm.at[nit_v.at[1]], nrows2_v.at[1], sem_n1)

        def jloop(it, _):
            jj = it * 2
            for p in range(2):
                j = jj + p
                wait(p)
                pool_chunk(j, p)

                @pl.when(j + 2 < NJ)
                def _():
                    issue(j + 2, p)
            return 0

        lax.fori_loop(0, NJ // 2, jloop, 0)
        pltpu.sync_copy(user_v, user_hbm.at[pl.ds(base, BPW)])

        pltpu.make_async_copy(table_hbm.at[pit_v], prows_v, sem_p).wait()
        pltpu.sync_copy(prows_v, posrows_hbm.at[pl.ds(base, BPW)])
        for q in range(NEG):
            pb = q % 2
            pltpu.make_async_copy(table_hbm.at[nit_v.at[q]], nrows2_v.at[pb],
                                  nsems[pb]).wait()
            pltpu.sync_copy(
                nrows2_v.at[pb],
                negrows_hbm.at[pl.ds(w * NEG * BPW + q * BPW, BPW)])
            if q + 2 < NEG:
                pltpu.async_copy(table_hbm.at[nit_v.at[q + 2]],
                                 nrows2_v.at[pb], nsems[pb])

    return k(seq_idx, pos_idx, neg_idx, table)


BM = 512
NB = B // BM


def _tc_body(u0_ref, w1_ref, b1_ref, w2_ref, b2_ref, w3_ref, b3_ref,
             pos_ref, neg_ref, out_ref):
    x = u0_ref[...]
    h = jnp.maximum(
        jnp.dot(x, w1_ref[...], preferred_element_type=jnp.float32)
        + b1_ref[...], 0.0)
    h = jnp.maximum(
        jnp.dot(h, w2_ref[...], preferred_element_type=jnp.float32)
        + b2_ref[...], 0.0)
    u = jnp.maximum(
        jnp.dot(h, w3_ref[...], preferred_element_type=jnp.float32)
        + b3_ref[...], 0.0)

    col = lax.broadcasted_iota(jnp.int32, (D, 1 + NEG), 1)
    s = jnp.dot(u * pos_ref[...], (col == 0).astype(jnp.float32),
                preferred_element_type=jnp.float32)
    for j in range(NEG):
        item = neg_ref[:, j * D:(j + 1) * D]
        s = s + jnp.dot(u * item, (col == (j + 1)).astype(jnp.float32),
                        preferred_element_type=jnp.float32)
    out_ref[...] = s


def _tc_mlp_scores(user0, W1, b1, W2, b2, W3, b3, pos_rows, neg_rows):
    return pl.pallas_call(
        _tc_body,
        grid=(NB,),
        in_specs=[
            pl.BlockSpec((BM, D), lambda i: (i, 0)),
            pl.BlockSpec((D, H1), lambda i: (0, 0)),
            pl.BlockSpec((1, H1), lambda i: (0, 0)),
            pl.BlockSpec((H1, H2), lambda i: (0, 0)),
            pl.BlockSpec((1, H2), lambda i: (0, 0)),
            pl.BlockSpec((H2, H3), lambda i: (0, 0)),
            pl.BlockSpec((1, H3), lambda i: (0, 0)),
            pl.BlockSpec((BM, D), lambda i: (i, 0)),
            pl.BlockSpec((BM, NEG * D), lambda i: (i, 0)),
        ],
        out_specs=pl.BlockSpec((BM, 1 + NEG), lambda i: (i, 0)),
        out_shape=jax.ShapeDtypeStruct((B, 1 + NEG), jnp.float32),
    )(user0, W1, b1.reshape(1, H1), W2, b2.reshape(1, H2),
      W3, b3.reshape(1, H3), pos_rows, neg_rows)


def kernel(click_seq, pos_item, neg_item, table, W1, b1, W2, b2, W3, b3):
    seq_idx = click_seq.astype(jnp.int32).reshape(JR, GIDX)
    pos_idx = pos_item.astype(jnp.int32).reshape(NW, BPW)
    neg_idx = neg_item.astype(jnp.int32).reshape(NW * NEG, BPW)
    user0, pos_rows, neg_rows = _sc_gather_pool(seq_idx, pos_idx, neg_idx,
                                                table)
    return _tc_mlp_scores(user0, W1, b1, W2, b2, W3, b3,
                          pos_rows, neg_rows.reshape(B, NEG * D))
